# Initial kernel scaffold; baseline (speedup 1.0000x reference)
import jax
import jax.numpy as jnp
from jax.experimental import pallas as pl


def kernel(x_movie, x_user, edge_index_um, edge_index_mu, W_in_movie, b_in_movie, W_in_user, b_in_user, W_out, b_out):
    raise NotImplementedError("write your pallas kernel here")


# speedup vs baseline: 10.3763x; 10.3763x over previous
import functools

import jax
import jax.numpy as jnp
from jax import lax
from jax.experimental import pallas as pl
from jax.experimental.pallas import tpu as pltpu
from jax.experimental.pallas import tpu_sc as plsc

N = 10000
E = 320000
D = 128
HID = 128
OUT = 32
ALPHA = 0.01

NC, NS = 2, 16
NW = NC * NS
EW = E // NW
CH = 80
NCHUNK = EW // CH
RPS = 624

_mesh = plsc.VectorSubcoreMesh(
    core_axis_name="c", subcore_axis_name="s", num_cores=NC, num_subcores=NS)


@functools.partial(
    pl.kernel,
    out_type=jax.ShapeDtypeStruct((NC, N, OUT), jnp.float32),
    mesh=_mesh,
    compiler_params=pltpu.CompilerParams(use_tc_tiling_on_sc=False),
    scratch_types=[
        pltpu.VMEM((NCHUNK, CH), jnp.int32),
        pltpu.VMEM((NCHUNK, CH), jnp.int32),
        pltpu.VMEM((CH, OUT), jnp.float32),
        pltpu.VMEM((RPS, OUT), jnp.float32),
        pltpu.VMEM_SHARED((N, OUT), jnp.float32),
        pltpu.SemaphoreType.DMA,
    ],
)
def _sc_segsum(table, src, dst, out, idx_s, idx_d, rows, zbuf, acc, sem):
    cid = lax.axis_index("c")
    sid = lax.axis_index("s")
    wid = cid * NS + sid

    z = jnp.zeros((16,), jnp.float32)

    def zb(j, carry):
        zbuf[j, pl.ds(0, 16)] = z
        zbuf[j, pl.ds(16, 16)] = z
        return carry

    lax.fori_loop(0, RPS, zb, 0)
    pltpu.sync_copy(zbuf, acc.at[pl.ds(sid * RPS, RPS)])

    @pl.when(sid == NS - 1)
    def _():
        pltpu.sync_copy(zbuf.at[pl.ds(0, 16)], acc.at[pl.ds(NS * RPS, 16)])

    plsc.subcore_barrier()

    pltpu.sync_copy(src.at[wid], idx_s)
    pltpu.sync_copy(dst.at[wid], idx_d)

    def eb(j, carry):
        pltpu.async_copy(table.at[idx_s.at[j]], rows, sem).wait()
        pltpu.sync_copy(rows, acc.at[idx_d.at[j]], add=True)
        return carry

    lax.fori_loop(0, NCHUNK, eb, 0)

    plsc.subcore_barrier()
    pltpu.sync_copy(acc.at[pl.ds(sid * RPS, RPS)],
                    out.at[cid, pl.ds(sid * RPS, RPS)])

    @pl.when(sid == NS - 1)
    def _():
        pltpu.sync_copy(acc.at[pl.ds(NS * RPS, 16)],
                        out.at[cid, pl.ds(NS * RPS, 16)])


@functools.partial(
    pl.kernel,
    out_type=jax.ShapeDtypeStruct((2, NW, N // 16, 16), jnp.float32),
    mesh=_mesh,
    compiler_params=pltpu.CompilerParams(needs_layout_passes=False),
    scratch_types=[
        pltpu.VMEM((NCHUNK, CH), jnp.int32),
        pltpu.VMEM((N // 16, 16), jnp.float32),
    ],
)
def _sc_degree(dst_um, dst_mu, out, idx_v, deg_v):
    cid = lax.axis_index("c")
    sid = lax.axis_index("s")
    wid = cid * NS + sid
    ones = jnp.ones((16,), jnp.float32)
    z = jnp.zeros((16,), jnp.float32)

    for e, dref in enumerate((dst_um, dst_mu)):
        def zb(j, carry):
            deg_v[j, pl.ds(0, 16)] = z
            return carry

        lax.fori_loop(0, N // 16, zb, 0)
        pltpu.sync_copy(dref.at[wid], idx_v)

        def hb(j, carry):
            def hb2(k, carry2):
                v = idx_v[j, pl.ds(k * 16, 16)]
                plsc.addupdate_scatter(
                    deg_v, [lax.shift_right_logical(v, 4),
                            lax.bitwise_and(v, 15)], ones)
                return carry2
            return lax.fori_loop(0, CH // 16, hb2, carry)

        lax.fori_loop(0, NCHUNK, hb, 0)
        pltpu.sync_copy(deg_v, out.at[e, wid])


BM = 1000


def _dense_body(x_ref, w1_ref, b1_ref, w2_ref, o_ref):
    h = jnp.dot(x_ref[...], w1_ref[0], preferred_element_type=jnp.float32)
    h = jnp.maximum(h + b1_ref[0], 0.0)
    o_ref[...] = jnp.dot(h, w2_ref[...], preferred_element_type=jnp.float32)


def _dense_proj(X, W1, B1, W2):
    nb = X.shape[0] // BM
    per = nb // 2
    return pl.pallas_call(
        _dense_body,
        grid=(nb,),
        in_specs=[
            pl.BlockSpec((BM, D), lambda i: (i, 0)),
            pl.BlockSpec((1, D, HID), lambda i: (i // per, 0, 0)),
            pl.BlockSpec((1, 1, HID), lambda i: (i // per, 0, 0)),
            pl.BlockSpec((HID, OUT), lambda i: (0, 0)),
        ],
        out_specs=pl.BlockSpec((BM, OUT), lambda i: (i, 0)),
        out_shape=jax.ShapeDtypeStruct((X.shape[0], OUT), jnp.float32),
    )(X, W1, B1, W2)


def _degsum_body(dp_ref, o_ref):
    o_ref[...] = jnp.maximum(jnp.sum(dp_ref[...], axis=1), 1.0)


def _degsum(degp):
    return pl.pallas_call(
        _degsum_body,
        grid=(1,),
        in_specs=[pl.BlockSpec((2, NW, N), lambda i: (0, 0, 0))],
        out_specs=pl.BlockSpec((2, N), lambda i: (0, 0)),
        out_shape=jax.ShapeDtypeStruct((2, N), jnp.float32),
    )(degp)


def _comb1_body(su_ref, d_ref, o_ref):
    o_ref[...] = (su_ref[0] + su_ref[1]) / d_ref[...]


def _combine1(su, deg):
    return pl.pallas_call(
        _comb1_body,
        grid=(N // BM,),
        in_specs=[
            pl.BlockSpec((2, BM, OUT), lambda i: (0, i, 0)),
            pl.BlockSpec((BM, 1), lambda i: (i, 0)),
        ],
        out_specs=pl.BlockSpec((BM, OUT), lambda i: (i, 0)),
        out_shape=jax.ShapeDtypeStruct((N, OUT), jnp.float32),
    )(su, deg)


def _final_body(g_ref, s1_ref, s2_ref, d_ref, b_ref, o_ref):
    num = (2.0 * ALPHA) * (s1_ref[0] + s1_ref[1]) + (s2_ref[0] + s2_ref[1])
    o_ref[...] = (ALPHA * ALPHA) * g_ref[...] + num / d_ref[...] + b_ref[...]


def _final(g_m, s1, s2, deg, b_out):
    return pl.pallas_call(
        _final_body,
        grid=(N // BM,),
        in_specs=[
            pl.BlockSpec((BM, OUT), lambda i: (i, 0)),
            pl.BlockSpec((2, BM, OUT), lambda i: (0, i, 0)),
            pl.BlockSpec((2, BM, OUT), lambda i: (0, i, 0)),
            pl.BlockSpec((BM, 1), lambda i: (i, 0)),
            pl.BlockSpec((1, OUT), lambda i: (0, 0)),
        ],
        out_specs=pl.BlockSpec((BM, OUT), lambda i: (i, 0)),
        out_shape=jax.ShapeDtypeStruct((N, OUT), jnp.float32),
    )(g_m, s1, s2, deg, b_out)


def kernel(x_movie, x_user, edge_index_um, edge_index_mu, W_in_movie,
           b_in_movie, W_in_user, b_in_user, W_out, b_out):
    X = jnp.concatenate([x_movie, x_user], axis=0)
    W1 = jnp.stack([W_in_movie, W_in_user])
    B1 = jnp.stack([b_in_movie, b_in_user]).reshape(2, 1, HID)
    G = _dense_proj(X, W1, B1, W_out)
    g_m, g_u = G[:N], G[N:]

    src_um = edge_index_um[0].astype(jnp.int32).reshape(NW, NCHUNK, CH)
    dst_um = edge_index_um[1].astype(jnp.int32).reshape(NW, NCHUNK, CH)
    src_mu = edge_index_mu[0].astype(jnp.int32).reshape(NW, NCHUNK, CH)
    dst_mu = edge_index_mu[1].astype(jnp.int32).reshape(NW, NCHUNK, CH)

    degp = _sc_degree(dst_um, dst_mu).reshape(2, NW, N)
    deg = _degsum(degp)
    deg_m = deg[0].reshape(N, 1)
    deg_u = deg[1].reshape(N, 1)
    su = _sc_segsum(g_m, src_mu, dst_mu)
    s1 = _sc_segsum(g_u, src_um, dst_um)
    t_u = _combine1(su, deg_u)
    s2 = _sc_segsum(t_u, src_um, dst_um)
    return _final(g_m, s1, s2, deg_m, b_out.reshape(1, OUT))


# speedup vs baseline: 11.6202x; 1.1199x over previous
import functools

import jax
import jax.numpy as jnp
from jax import lax
from jax.experimental import pallas as pl
from jax.experimental.pallas import tpu as pltpu
from jax.experimental.pallas import tpu_sc as plsc

N = 10000
E = 320000
D = 128
HID = 128
OUT = 32
ALPHA = 0.01

NC, NS = 2, 16
NW = NC * NS
EW = E // NW
CH = 80
NCHUNK = EW // CH
RPS = 624

_mesh = plsc.VectorSubcoreMesh(
    core_axis_name="c", subcore_axis_name="s", num_cores=NC, num_subcores=NS)


@functools.partial(
    pl.kernel,
    out_type=jax.ShapeDtypeStruct((NC, N, OUT), jnp.float32),
    mesh=_mesh,
    compiler_params=pltpu.CompilerParams(use_tc_tiling_on_sc=False),
    scratch_types=[
        pltpu.VMEM((NCHUNK, CH), jnp.int32),
        pltpu.VMEM((NCHUNK, CH), jnp.int32),
        pltpu.VMEM((CH, OUT), jnp.float32),
        pltpu.VMEM((CH, OUT), jnp.float32),
        pltpu.VMEM((RPS, OUT), jnp.float32),
        pltpu.VMEM_SHARED((N, OUT), jnp.float32),
        pltpu.SemaphoreType.DMA,
        pltpu.SemaphoreType.DMA,
        pltpu.SemaphoreType.DMA,
        pltpu.SemaphoreType.DMA,
    ],
)
def _sc_segsum(table, src, dst, out, idx_s, idx_d, rows0, rows1, zbuf, acc,
               gsem0, gsem1, ssem0, ssem1):
    cid = lax.axis_index("c")
    sid = lax.axis_index("s")
    wid = cid * NS + sid

    z = jnp.zeros((16,), jnp.float32)

    def zb(j, carry):
        zbuf[j, pl.ds(0, 16)] = z
        zbuf[j, pl.ds(16, 16)] = z
        return carry

    lax.fori_loop(0, RPS, zb, 0)
    pltpu.sync_copy(zbuf, acc.at[pl.ds(sid * RPS, RPS)])

    @pl.when(sid == NS - 1)
    def _():
        pltpu.sync_copy(zbuf.at[pl.ds(0, 16)], acc.at[pl.ds(NS * RPS, 16)])

    plsc.subcore_barrier()

    pltpu.sync_copy(src.at[wid], idx_s)
    pltpu.sync_copy(dst.at[wid], idx_d)

    def step(j, rows, gsem, ssem, nrows, ngsem, nssem):
        pltpu.make_async_copy(table.at[idx_s.at[j]], rows, gsem).wait()
        pltpu.async_copy(rows, acc.at[idx_d.at[j]], ssem, add=True)

        @pl.when(j + 1 < NCHUNK)
        def _():
            @pl.when(j >= 1)
            def _():
                pltpu.make_async_copy(
                    nrows, acc.at[idx_d.at[j - 1]], nssem).wait()
            pltpu.async_copy(table.at[idx_s.at[j + 1]], nrows, ngsem)

    def eb(j, carry):
        @pl.when(j % 2 == 0)
        def _():
            step(j, rows0, gsem0, ssem0, rows1, gsem1, ssem1)

        @pl.when(j % 2 == 1)
        def _():
            step(j, rows1, gsem1, ssem1, rows0, gsem0, ssem0)

        return carry

    pltpu.async_copy(table.at[idx_s.at[0]], rows0, gsem0)
    lax.fori_loop(0, NCHUNK, eb, 0)
    pltpu.make_async_copy(rows1, acc.at[idx_d.at[NCHUNK - 2]], ssem1).wait()
    pltpu.make_async_copy(rows0, acc.at[idx_d.at[NCHUNK - 1]], ssem0).wait()

    plsc.subcore_barrier()
    pltpu.sync_copy(acc.at[pl.ds(sid * RPS, RPS)],
                    out.at[cid, pl.ds(sid * RPS, RPS)])

    @pl.when(sid == NS - 1)
    def _():
        pltpu.sync_copy(acc.at[pl.ds(NS * RPS, 16)],
                        out.at[cid, pl.ds(NS * RPS, 16)])


@functools.partial(
    pl.kernel,
    out_type=jax.ShapeDtypeStruct((2, NW, N // 16, 16), jnp.float32),
    mesh=_mesh,
    compiler_params=pltpu.CompilerParams(needs_layout_passes=False),
    scratch_types=[
        pltpu.VMEM((NCHUNK, CH), jnp.int32),
        pltpu.VMEM((N // 16, 16), jnp.float32),
    ],
)
def _sc_degree(dst_um, dst_mu, out, idx_v, deg_v):
    cid = lax.axis_index("c")
    sid = lax.axis_index("s")
    wid = cid * NS + sid
    ones = jnp.ones((16,), jnp.float32)
    z = jnp.zeros((16,), jnp.float32)

    for e, dref in enumerate((dst_um, dst_mu)):
        def zb(j, carry):
            deg_v[j, pl.ds(0, 16)] = z
            return carry

        lax.fori_loop(0, N // 16, zb, 0)
        pltpu.sync_copy(dref.at[wid], idx_v)

        def hb(j, carry):
            def hb2(k, carry2):
                v = idx_v[j, pl.ds(k * 16, 16)]
                plsc.addupdate_scatter(
                    deg_v, [lax.shift_right_logical(v, 4),
                            lax.bitwise_and(v, 15)], ones)
                return carry2
            return lax.fori_loop(0, CH // 16, hb2, carry)

        lax.fori_loop(0, NCHUNK, hb, 0)
        pltpu.sync_copy(deg_v, out.at[e, wid])


BM = 1000


def _dense_body(x_ref, w1_ref, b1_ref, w2_ref, o_ref):
    h = jnp.dot(x_ref[...], w1_ref[0], preferred_element_type=jnp.float32)
    h = jnp.maximum(h + b1_ref[0], 0.0)
    o_ref[...] = jnp.dot(h, w2_ref[...], preferred_element_type=jnp.float32)


def _dense_proj(X, W1, B1, W2):
    nb = X.shape[0] // BM
    per = nb // 2
    return pl.pallas_call(
        _dense_body,
        grid=(nb,),
        in_specs=[
            pl.BlockSpec((BM, D), lambda i: (i, 0)),
            pl.BlockSpec((1, D, HID), lambda i: (i // per, 0, 0)),
            pl.BlockSpec((1, 1, HID), lambda i: (i // per, 0, 0)),
            pl.BlockSpec((HID, OUT), lambda i: (0, 0)),
        ],
        out_specs=pl.BlockSpec((BM, OUT), lambda i: (i, 0)),
        out_shape=jax.ShapeDtypeStruct((X.shape[0], OUT), jnp.float32),
    )(X, W1, B1, W2)


def _degsum_body(dp_ref, o_ref):
    o_ref[...] = jnp.maximum(jnp.sum(dp_ref[...], axis=1), 1.0)


def _degsum(degp):
    return pl.pallas_call(
        _degsum_body,
        grid=(1,),
        in_specs=[pl.BlockSpec((2, NW, N), lambda i: (0, 0, 0))],
        out_specs=pl.BlockSpec((2, N), lambda i: (0, 0)),
        out_shape=jax.ShapeDtypeStruct((2, N), jnp.float32),
    )(degp)


def _comb1_body(su_ref, d_ref, o_ref):
    o_ref[...] = (su_ref[0] + su_ref[1]) / d_ref[...]


def _combine1(su, deg):
    return pl.pallas_call(
        _comb1_body,
        grid=(N // BM,),
        in_specs=[
            pl.BlockSpec((2, BM, OUT), lambda i: (0, i, 0)),
            pl.BlockSpec((BM, 1), lambda i: (i, 0)),
        ],
        out_specs=pl.BlockSpec((BM, OUT), lambda i: (i, 0)),
        out_shape=jax.ShapeDtypeStruct((N, OUT), jnp.float32),
    )(su, deg)


def _final_body(g_ref, s1_ref, s2_ref, d_ref, b_ref, o_ref):
    num = (2.0 * ALPHA) * (s1_ref[0] + s1_ref[1]) + (s2_ref[0] + s2_ref[1])
    o_ref[...] = (ALPHA * ALPHA) * g_ref[...] + num / d_ref[...] + b_ref[...]


def _final(g_m, s1, s2, deg, b_out):
    return pl.pallas_call(
        _final_body,
        grid=(N // BM,),
        in_specs=[
            pl.BlockSpec((BM, OUT), lambda i: (i, 0)),
            pl.BlockSpec((2, BM, OUT), lambda i: (0, i, 0)),
            pl.BlockSpec((2, BM, OUT), lambda i: (0, i, 0)),
            pl.BlockSpec((BM, 1), lambda i: (i, 0)),
            pl.BlockSpec((1, OUT), lambda i: (0, 0)),
        ],
        out_specs=pl.BlockSpec((BM, OUT), lambda i: (i, 0)),
        out_shape=jax.ShapeDtypeStruct((N, OUT), jnp.float32),
    )(g_m, s1, s2, deg, b_out)


def kernel(x_movie, x_user, edge_index_um, edge_index_mu, W_in_movie,
           b_in_movie, W_in_user, b_in_user, W_out, b_out):
    X = jnp.concatenate([x_movie, x_user], axis=0)
    W1 = jnp.stack([W_in_movie, W_in_user])
    B1 = jnp.stack([b_in_movie, b_in_user]).reshape(2, 1, HID)
    G = _dense_proj(X, W1, B1, W_out)
    g_m, g_u = G[:N], G[N:]

    src_um = edge_index_um[0].astype(jnp.int32).reshape(NW, NCHUNK, CH)
    dst_um = edge_index_um[1].astype(jnp.int32).reshape(NW, NCHUNK, CH)
    src_mu = edge_index_mu[0].astype(jnp.int32).reshape(NW, NCHUNK, CH)
    dst_mu = edge_index_mu[1].astype(jnp.int32).reshape(NW, NCHUNK, CH)

    degp = _sc_degree(dst_um, dst_mu).reshape(2, NW, N)
    deg = _degsum(degp)
    deg_m = deg[0].reshape(N, 1)
    deg_u = deg[1].reshape(N, 1)
    su = _sc_segsum(g_m, src_mu, dst_mu)
    s1 = _sc_segsum(g_u, src_um, dst_um)
    t_u = _combine1(su, deg_u)
    s2 = _sc_segsum(t_u, src_um, dst_um)
    return _final(g_m, s1, s2, deg_m, b_out.reshape(1, OUT))


# speedup vs baseline: 18.2896x; 1.5739x over previous
import functools

import jax
import jax.numpy as jnp
from jax import lax
from jax.experimental import pallas as pl
from jax.experimental.pallas import tpu as pltpu
from jax.experimental.pallas import tpu_sc as plsc

N = 10000
E = 320000
D = 128
HID = 128
OUT = 32
ALPHA = 0.01

NC, NS = 2, 16
NW = NC * NS
EW = E // NW
CH = 80
NCHUNK = EW // CH
RPS = 624

_mesh = plsc.VectorSubcoreMesh(
    core_axis_name="c", subcore_axis_name="s", num_cores=NC, num_subcores=NS)


@functools.partial(
    pl.kernel,
    out_type=jax.ShapeDtypeStruct((NC, N, OUT), jnp.float32),
    mesh=_mesh,
    compiler_params=pltpu.CompilerParams(use_tc_tiling_on_sc=False),
    scratch_types=[
        pltpu.VMEM((NCHUNK, CH), jnp.int32),
        pltpu.VMEM((NCHUNK, CH), jnp.int32),
        [pltpu.VMEM((CH, OUT), jnp.float32)] * 4,
        pltpu.VMEM((RPS, OUT), jnp.float32),
        pltpu.VMEM_SHARED((N, OUT), jnp.float32),
        [pltpu.SemaphoreType.DMA] * 4,
        [pltpu.SemaphoreType.DMA] * 4,
    ],
)
def _sc_segsum(table, src, dst, out, idx_s, idx_d, rows, zbuf, acc,
               gsems, ssems):
    cid = lax.axis_index("c")
    sid = lax.axis_index("s")
    wid = cid * NS + sid

    z = jnp.zeros((16,), jnp.float32)

    def zb(j, carry):
        zbuf[j, pl.ds(0, 16)] = z
        zbuf[j, pl.ds(16, 16)] = z
        return carry

    lax.fori_loop(0, RPS, zb, 0)
    pltpu.sync_copy(zbuf, acc.at[pl.ds(sid * RPS, RPS)])

    @pl.when(sid == NS - 1)
    def _():
        pltpu.sync_copy(zbuf.at[pl.ds(0, 16)], acc.at[pl.ds(NS * RPS, 16)])

    plsc.subcore_barrier()

    pltpu.sync_copy(src.at[wid], idx_s)
    pltpu.sync_copy(dst.at[wid], idx_d)

    NB = 4

    def step(j, b):
        bn = (b + NB - 1) % NB
        pltpu.make_async_copy(table.at[idx_s.at[j]], rows[b], gsems[b]).wait()
        pltpu.async_copy(rows[b], acc.at[idx_d.at[j]], ssems[b], add=True)

        @pl.when(j + NB - 1 < NCHUNK)
        def _():
            @pl.when(j >= 1)
            def _():
                pltpu.make_async_copy(
                    rows[bn], acc.at[idx_d.at[j - 1]], ssems[bn]).wait()
            pltpu.async_copy(table.at[idx_s.at[j + NB - 1]], rows[bn],
                             gsems[bn])

    def eb(j, carry):
        for b in range(NB):
            @pl.when(j % NB == b)
            def _(b=b):
                step(j, b)
        return carry

    for k in range(NB - 1):
        pltpu.async_copy(table.at[idx_s.at[k]], rows[k], gsems[k])
    lax.fori_loop(0, NCHUNK, eb, 0)
    for j in range(NCHUNK - NB, NCHUNK):
        b = j % NB
        pltpu.make_async_copy(rows[b], acc.at[idx_d.at[j]], ssems[b]).wait()

    plsc.subcore_barrier()
    pltpu.sync_copy(acc.at[pl.ds(sid * RPS, RPS)],
                    out.at[cid, pl.ds(sid * RPS, RPS)])

    @pl.when(sid == NS - 1)
    def _():
        pltpu.sync_copy(acc.at[pl.ds(NS * RPS, 16)],
                        out.at[cid, pl.ds(NS * RPS, 16)])


@functools.partial(
    pl.kernel,
    out_type=jax.ShapeDtypeStruct((2, NW, N // 16, 16), jnp.float32),
    mesh=_mesh,
    compiler_params=pltpu.CompilerParams(needs_layout_passes=False),
    scratch_types=[
        pltpu.VMEM((NCHUNK, CH), jnp.int32),
        pltpu.VMEM((N // 16, 16), jnp.float32),
    ],
)
def _sc_degree(dst_um, dst_mu, out, idx_v, deg_v):
    cid = lax.axis_index("c")
    sid = lax.axis_index("s")
    wid = cid * NS + sid
    ones = jnp.ones((16,), jnp.float32)
    z = jnp.zeros((16,), jnp.float32)

    for e, dref in enumerate((dst_um, dst_mu)):
        def zb(j, carry):
            deg_v[j, pl.ds(0, 16)] = z
            return carry

        lax.fori_loop(0, N // 16, zb, 0)
        pltpu.sync_copy(dref.at[wid], idx_v)

        def hb(j, carry):
            def hb2(k, carry2):
                v = idx_v[j, pl.ds(k * 16, 16)]
                plsc.addupdate_scatter(
                    deg_v, [lax.shift_right_logical(v, 4),
                            lax.bitwise_and(v, 15)], ones)
                return carry2
            return lax.fori_loop(0, CH // 16, hb2, carry)

        lax.fori_loop(0, NCHUNK, hb, 0)
        pltpu.sync_copy(deg_v, out.at[e, wid])


BM = 1000


def _dense_body(x_ref, w1_ref, b1_ref, w2_ref, o_ref):
    h = jnp.dot(x_ref[...], w1_ref[0], preferred_element_type=jnp.float32)
    h = jnp.maximum(h + b1_ref[0], 0.0)
    o_ref[...] = jnp.dot(h, w2_ref[...], preferred_element_type=jnp.float32)


def _dense_proj(X, W1, B1, W2):
    nb = X.shape[0] // BM
    per = nb // 2
    return pl.pallas_call(
        _dense_body,
        grid=(nb,),
        in_specs=[
            pl.BlockSpec((BM, D), lambda i: (i, 0)),
            pl.BlockSpec((1, D, HID), lambda i: (i // per, 0, 0)),
            pl.BlockSpec((1, 1, HID), lambda i: (i // per, 0, 0)),
            pl.BlockSpec((HID, OUT), lambda i: (0, 0)),
        ],
        out_specs=pl.BlockSpec((BM, OUT), lambda i: (i, 0)),
        out_shape=jax.ShapeDtypeStruct((X.shape[0], OUT), jnp.float32),
    )(X, W1, B1, W2)


def _degsum_body(dp_ref, o_ref):
    o_ref[...] = jnp.maximum(jnp.sum(dp_ref[...], axis=1), 1.0)


def _degsum(degp):
    return pl.pallas_call(
        _degsum_body,
        grid=(1,),
        in_specs=[pl.BlockSpec((2, NW, N), lambda i: (0, 0, 0))],
        out_specs=pl.BlockSpec((2, N), lambda i: (0, 0)),
        out_shape=jax.ShapeDtypeStruct((2, N), jnp.float32),
    )(degp)


def _comb1_body(su_ref, d_ref, o_ref):
    o_ref[...] = (su_ref[0] + su_ref[1]) / d_ref[...]


def _combine1(su, deg):
    return pl.pallas_call(
        _comb1_body,
        grid=(N // BM,),
        in_specs=[
            pl.BlockSpec((2, BM, OUT), lambda i: (0, i, 0)),
            pl.BlockSpec((BM, 1), lambda i: (i, 0)),
        ],
        out_specs=pl.BlockSpec((BM, OUT), lambda i: (i, 0)),
        out_shape=jax.ShapeDtypeStruct((N, OUT), jnp.float32),
    )(su, deg)


def _final_body(g_ref, s1_ref, s2_ref, d_ref, b_ref, o_ref):
    num = (2.0 * ALPHA) * (s1_ref[0] + s1_ref[1]) + (s2_ref[0] + s2_ref[1])
    o_ref[...] = (ALPHA * ALPHA) * g_ref[...] + num / d_ref[...] + b_ref[...]


def _final(g_m, s1, s2, deg, b_out):
    return pl.pallas_call(
        _final_body,
        grid=(N // BM,),
        in_specs=[
            pl.BlockSpec((BM, OUT), lambda i: (i, 0)),
            pl.BlockSpec((2, BM, OUT), lambda i: (0, i, 0)),
            pl.BlockSpec((2, BM, OUT), lambda i: (0, i, 0)),
            pl.BlockSpec((BM, 1), lambda i: (i, 0)),
            pl.BlockSpec((1, OUT), lambda i: (0, 0)),
        ],
        out_specs=pl.BlockSpec((BM, OUT), lambda i: (i, 0)),
        out_shape=jax.ShapeDtypeStruct((N, OUT), jnp.float32),
    )(g_m, s1, s2, deg, b_out)


def kernel(x_movie, x_user, edge_index_um, edge_index_mu, W_in_movie,
           b_in_movie, W_in_user, b_in_user, W_out, b_out):
    X = jnp.concatenate([x_movie, x_user], axis=0)
    W1 = jnp.stack([W_in_movie, W_in_user])
    B1 = jnp.stack([b_in_movie, b_in_user]).reshape(2, 1, HID)
    G = _dense_proj(X, W1, B1, W_out)
    g_m, g_u = G[:N], G[N:]

    src_um = edge_index_um[0].astype(jnp.int32).reshape(NW, NCHUNK, CH)
    dst_um = edge_index_um[1].astype(jnp.int32).reshape(NW, NCHUNK, CH)
    src_mu = edge_index_mu[0].astype(jnp.int32).reshape(NW, NCHUNK, CH)
    dst_mu = edge_index_mu[1].astype(jnp.int32).reshape(NW, NCHUNK, CH)

    degp = _sc_degree(dst_um, dst_mu).reshape(2, NW, N)
    deg = _degsum(degp)
    deg_m = deg[0].reshape(N, 1)
    deg_u = deg[1].reshape(N, 1)
    su = _sc_segsum(g_m, src_mu, dst_mu)
    s1 = _sc_segsum(g_u, src_um, dst_um)
    t_u = _combine1(su, deg_u)
    s2 = _sc_segsum(t_u, src_um, dst_um)
    return _final(g_m, s1, s2, deg_m, b_out.reshape(1, OUT))
